# Initial kernel scaffold; baseline (speedup 1.0000x reference)
import jax
import jax.numpy as jnp
from jax.experimental import pallas as pl


def kernel(x, table):
    raise NotImplementedError("write your pallas kernel here")


# speedup vs baseline: 1.5748x; 1.5748x over previous
import functools

import jax
import jax.numpy as jnp
from jax import lax
from jax.experimental import pallas as pl
from jax.experimental.pallas import tpu as pltpu
from jax.experimental.pallas import tpu_sc as plsc

_VOCAB = 1000000
_EMBED = 64
_BATCH = 16384
_HIST = 50
_B = _BATCH * _HIST

_NC = 2
_NS = 16
_NW = _NC * _NS
_B_PER_W = _B // _NW
_CHUNK = 128
_NCHUNK = _B_PER_W // _CHUNK

_mesh = plsc.VectorSubcoreMesh(core_axis_name="c", subcore_axis_name="s")


@functools.partial(
    pl.kernel,
    mesh=_mesh,
    out_type=jax.ShapeDtypeStruct((_B, _EMBED), jnp.float32),
    scratch_types=[
        pltpu.VMEM((_CHUNK,), jnp.int32),
        pltpu.VMEM((_CHUNK, _EMBED), jnp.float32),
        pltpu.SemaphoreType.DMA,
    ],
    compiler_params=pltpu.CompilerParams(use_tc_tiling_on_sc=False),
)
def _gather_sc(idx_hbm, table_hbm, out_hbm, idx_v, rows_v, sem):
    wid = lax.axis_index("s") * _NC + lax.axis_index("c")
    base = wid * _B_PER_W

    def body(g, carry):
        off = base + g * _CHUNK
        pltpu.sync_copy(idx_hbm.at[pl.ds(off, _CHUNK)], idx_v)
        pltpu.async_copy(table_hbm.at[idx_v], rows_v, sem).wait()
        pltpu.sync_copy(rows_v, out_hbm.at[pl.ds(off, _CHUNK)])
        return carry

    lax.fori_loop(0, _NCHUNK, body, 0)


def kernel(x, table):
    flat = x.reshape(-1).astype(jnp.int32)
    out = _gather_sc(flat, table)
    return out.reshape(_BATCH, _HIST, _EMBED)


# speedup vs baseline: 1.8741x; 1.1901x over previous
import functools

import jax
import jax.numpy as jnp
from jax import lax
from jax.experimental import pallas as pl
from jax.experimental.pallas import tpu as pltpu
from jax.experimental.pallas import tpu_sc as plsc

_VOCAB = 1000000
_EMBED = 64
_BATCH = 16384
_HIST = 50
_B = _BATCH * _HIST

_NC = 2
_NS = 16
_NW = _NC * _NS
_B_PER_W = _B // _NW
_CHUNK = 256
_NCHUNK = _B_PER_W // _CHUNK
_NBUF = 4

_mesh = plsc.VectorSubcoreMesh(core_axis_name="c", subcore_axis_name="s")


@functools.partial(
    pl.kernel,
    mesh=_mesh,
    out_type=jax.ShapeDtypeStruct((_B, _EMBED), jnp.float32),
    scratch_types=[
        pltpu.VMEM((_B_PER_W,), jnp.int32),
        pltpu.VMEM((_NBUF, _CHUNK, _EMBED), jnp.float32),
        pltpu.SemaphoreType.DMA,
        pltpu.SemaphoreType.DMA,
        pltpu.SemaphoreType.DMA,
        pltpu.SemaphoreType.DMA,
        pltpu.SemaphoreType.DMA,
        pltpu.SemaphoreType.DMA,
        pltpu.SemaphoreType.DMA,
        pltpu.SemaphoreType.DMA,
    ],
    compiler_params=pltpu.CompilerParams(use_tc_tiling_on_sc=False),
)
def _gather_sc(idx_hbm, table_hbm, out_hbm, idx_v, rows_v,
               g0, g1, g2, g3, s0, s1, s2, s3):
    gsem = (g0, g1, g2, g3)
    ssem = (s0, s1, s2, s3)
    wid = lax.axis_index("s") * _NC + lax.axis_index("c")
    base = wid * _B_PER_W

    pltpu.sync_copy(idx_hbm.at[pl.ds(base, _B_PER_W)], idx_v)

    def idx_slice(g):
        return idx_v.at[pl.ds(g * _CHUNK, _CHUNK)]

    def out_slice(g):
        return out_hbm.at[pl.ds(base + g * _CHUNK, _CHUNK)]

    def issue_gather(g, b):
        pltpu.async_copy(table_hbm.at[idx_slice(g)], rows_v.at[b], gsem[b])

    def wait_gather(g, b):
        pltpu.make_async_copy(table_hbm.at[idx_slice(g)], rows_v.at[b],
                              gsem[b]).wait()

    def issue_store(g, b):
        pltpu.async_copy(rows_v.at[b], out_slice(g), ssem[b])

    def wait_store(g, b):
        pltpu.make_async_copy(rows_v.at[b], out_slice(g), ssem[b]).wait()

    issue_gather(0, 0)
    issue_gather(1, 1)
    wait_gather(0, 0)
    issue_store(0, 0)
    issue_gather(2, 2)
    wait_gather(1, 1)
    issue_store(1, 1)
    issue_gather(3, 3)

    def group(gg, carry):
        for k in range(_NBUF):
            b = (2 + k) % _NBUF
            g = 2 + gg * _NBUF + k
            wait_gather(g, b)
            issue_store(g, b)
            bb = (b + 2) % _NBUF
            wait_store(g - 2, bb)
            issue_gather(g + 2, bb)
        return carry

    lax.fori_loop(0, (_NCHUNK - 4) // _NBUF, group, 0)

    wait_gather(_NCHUNK - 2, (_NCHUNK - 2) % _NBUF)
    issue_store(_NCHUNK - 2, (_NCHUNK - 2) % _NBUF)
    wait_gather(_NCHUNK - 1, (_NCHUNK - 1) % _NBUF)
    issue_store(_NCHUNK - 1, (_NCHUNK - 1) % _NBUF)
    for g in range(_NCHUNK - 4, _NCHUNK):
        wait_store(g, g % _NBUF)


def kernel(x, table):
    flat = x.reshape(-1).astype(jnp.int32)
    out = _gather_sc(flat, table)
    return out.reshape(_BATCH, _HIST, _EMBED)
